# Initial kernel scaffold; baseline (speedup 1.0000x reference)
import jax
import jax.numpy as jnp
from jax.experimental import pallas as pl


def kernel(x, W):
    raise NotImplementedError("write your pallas kernel here")


# speedup vs baseline: 1.3907x; 1.3907x over previous
import functools
import math

import jax
import jax.numpy as jnp
import numpy as np
from jax import lax
from jax.experimental import pallas as pl
from jax.experimental.pallas import tpu as pltpu
from jax.experimental.pallas import tpu_sc as plsc

VOCAB = 1000000
DIM = 32
MAX_LEN = 200
BATCH = 4096
SEQ = 200

NC, NS = 2, 16
NW = NC * NS
N = BATCH * SEQ
PER_W = N // NW
SUB = 100
SEQ_PER_CHUNK = 4
CHUNK = SEQ_PER_CHUNK * SEQ
SUBS_PER_CHUNK = CHUNK // SUB
CHUNKS = PER_W // CHUNK
SCALE = math.sqrt(DIM)


def _make_pos_table():
    para = np.arange(MAX_LEN).reshape(-1, 1) / np.power(
        10000.0, np.arange(0, DIM, 2) / DIM)
    pos = np.zeros((MAX_LEN, DIM), dtype=np.float32)
    pos[:, 0::2] = np.sin(para)
    pos[:, 1::2] = np.cos(para)
    return pos


_MESH = plsc.VectorSubcoreMesh(core_axis_name="c", subcore_axis_name="s",
                               num_cores=NC, num_subcores=NS)


@functools.partial(
    pl.kernel,
    out_type=jax.ShapeDtypeStruct((N, DIM), jnp.float32),
    mesh=_MESH,
    compiler_params=pltpu.CompilerParams(use_tc_tiling_on_sc=False),
    scratch_types=[
        pltpu.VMEM((SUBS_PER_CHUNK, SUB), jnp.int32),
        pltpu.VMEM((CHUNK, DIM), jnp.float32),
        pltpu.VMEM((SEQ, DIM), jnp.float32),
        pltpu.SemaphoreType.DMA,
    ],
)
def _sc_embed(w_hbm, x_hbm, pos_hbm, out_hbm, idx_v, rows_v, pos_v, sem):
    wid = lax.axis_index("s") * NC + lax.axis_index("c")
    pltpu.sync_copy(pos_hbm, pos_v)

    def chunk_body(ci, _):
        base = pl.multiple_of(wid * PER_W + ci * CHUNK, CHUNK)
        base_row = pl.multiple_of(
            wid * (PER_W // SUB) + ci * SUBS_PER_CHUNK, SUBS_PER_CHUNK)
        pltpu.sync_copy(x_hbm.at[pl.ds(base_row, SUBS_PER_CHUNK)], idx_v)
        for j in range(SUBS_PER_CHUNK):
            pltpu.async_copy(w_hbm.at[idx_v.at[j]],
                             rows_v.at[pl.ds(j * SUB, SUB)], sem)
        for j in range(SUBS_PER_CHUNK):
            pltpu.make_async_copy(w_hbm.at[idx_v.at[j]],
                                  rows_v.at[pl.ds(j * SUB, SUB)], sem).wait()

        def pos_body(l, _):
            p0 = pos_v[l, pl.ds(0, 16)]
            p1 = pos_v[l, pl.ds(16, 16)]
            for s in range(SEQ_PER_CHUNK):
                i = s * SEQ + l
                rows_v[i, pl.ds(0, 16)] = rows_v[i, pl.ds(0, 16)] * SCALE + p0
                rows_v[i, pl.ds(16, 16)] = rows_v[i, pl.ds(16, 16)] * SCALE + p1
            return 0

        lax.fori_loop(0, SEQ, pos_body, 0)
        pltpu.sync_copy(rows_v, out_hbm.at[pl.ds(base, CHUNK)])
        return 0

    lax.fori_loop(0, CHUNKS, chunk_body, 0)


def kernel(x, W):
    pos = jnp.asarray(_make_pos_table())
    x2d = x.reshape(N // SUB, SUB)
    out = _sc_embed(W, x2d, pos)
    return out.reshape(BATCH, SEQ, DIM)


# speedup vs baseline: 1.4409x; 1.0361x over previous
import functools
import math

import jax
import jax.numpy as jnp
import numpy as np
from jax import lax
from jax.experimental import pallas as pl
from jax.experimental.pallas import tpu as pltpu
from jax.experimental.pallas import tpu_sc as plsc

VOCAB = 1000000
DIM = 32
MAX_LEN = 200
BATCH = 4096
SEQ = 200

NC, NS = 2, 16
NW = NC * NS
N = BATCH * SEQ
PER_W = N // NW
SUB = 100
SEQ_PER_CHUNK = 2
CHUNK = SEQ_PER_CHUNK * SEQ
SUBS_PER_CHUNK = CHUNK // SUB
CHUNKS = PER_W // CHUNK
IDX_ROWS = PER_W // SUB
NBUF = 4
SCALE = math.sqrt(DIM)


def _make_pos_table():
    para = np.arange(MAX_LEN).reshape(-1, 1) / np.power(
        10000.0, np.arange(0, DIM, 2) / DIM)
    pos = np.zeros((MAX_LEN, DIM), dtype=np.float32)
    pos[:, 0::2] = np.sin(para)
    pos[:, 1::2] = np.cos(para)
    return pos


_MESH = plsc.VectorSubcoreMesh(core_axis_name="c", subcore_axis_name="s",
                               num_cores=NC, num_subcores=NS)


@functools.partial(
    pl.kernel,
    out_type=jax.ShapeDtypeStruct((N, DIM), jnp.float32),
    mesh=_MESH,
    compiler_params=pltpu.CompilerParams(use_tc_tiling_on_sc=False),
    scratch_types=[
        pltpu.VMEM((IDX_ROWS, SUB), jnp.int32),
        pltpu.VMEM((SEQ, DIM), jnp.float32),
        [pltpu.VMEM((CHUNK, DIM), jnp.float32)] * NBUF,
        [pltpu.SemaphoreType.DMA] * NBUF,
        [pltpu.SemaphoreType.DMA] * NBUF,
    ],
)
def _sc_embed(w_hbm, x_hbm, pos_hbm, out_hbm, idx_all, pos_v, rows, gsem, ssem):
    wid = lax.axis_index("s") * NC + lax.axis_index("c")
    pltpu.sync_copy(pos_hbm, pos_v)
    idx_base = pl.multiple_of(wid * IDX_ROWS, IDX_ROWS)
    pltpu.sync_copy(x_hbm.at[pl.ds(idx_base, IDX_ROWS)], idx_all)
    out_base = wid * PER_W

    def fire_gather(g, b):
        r0 = g * SUBS_PER_CHUNK
        for j in range(SUBS_PER_CHUNK):
            pltpu.async_copy(w_hbm.at[idx_all.at[r0 + j]],
                             rows[b].at[pl.ds(j * SUB, SUB)], gsem[b])

    def drain_gather(b):
        pltpu.make_async_copy(w_hbm.at[pl.ds(0, CHUNK)], rows[b],
                              gsem[b]).wait()

    def wait_store(b):
        pltpu.make_async_copy(rows[b], w_hbm.at[pl.ds(0, CHUNK)],
                              ssem[b]).wait()

    def compute(b):
        rb = rows[b]

        def pos_body(l, _):
            p0 = pos_v[l, pl.ds(0, 16)]
            p1 = pos_v[l, pl.ds(16, 16)]
            for s in range(SEQ_PER_CHUNK):
                i = s * SEQ + l
                rb[i, pl.ds(0, 16)] = rb[i, pl.ds(0, 16)] * SCALE + p0
                rb[i, pl.ds(16, 16)] = rb[i, pl.ds(16, 16)] * SCALE + p1
            return 0

        lax.fori_loop(0, SEQ, pos_body, 0)

    for g in range(NBUF - 1):
        fire_gather(g, g)

    def ring_body(p, _):
        for b in range(NBUF):
            g = p * NBUF + b
            b2 = (b + NBUF - 1) % NBUF

            @pl.when(g + NBUF - 1 < CHUNKS)
            def _(b=b, b2=b2, g=g):
                if b == 0:
                    @pl.when(g > 0)
                    def _():
                        wait_store(b2)
                else:
                    wait_store(b2)
                fire_gather(g + NBUF - 1, b2)

            drain_gather(b)
            compute(b)
            pltpu.async_copy(
                rows[b],
                out_hbm.at[pl.ds(pl.multiple_of(out_base + g * CHUNK, CHUNK),
                                 CHUNK)],
                ssem[b])
        return 0

    lax.fori_loop(0, CHUNKS // NBUF, ring_body, 0)
    for b in range(NBUF):
        wait_store(b)


def kernel(x, W):
    pos = jnp.asarray(_make_pos_table())
    x2d = x.reshape(N // SUB, SUB)
    out = _sc_embed(W, x2d, pos)
    return out.reshape(BATCH, SEQ, DIM)
